# baseline (speedup 1.0000x reference)
import functools

import jax
import jax.numpy as jnp
from jax import lax
from jax.experimental import pallas as pl
from jax.experimental.pallas import tpu as pltpu
from jax.experimental.pallas import tpu_sc as plsc

_DIM = 64
_NC = 2
_NS = 16
_NW = _NC * _NS
_CHUNK = 1024


def _emb_body(table_hbm, idx_hbm, out_hbm, idx_v, rows_v, sem):
    wid = lax.axis_index("s") * _NC + lax.axis_index("c")
    b_per_w = idx_hbm.shape[0] // _NW
    base = wid * b_per_w
    nchunks = b_per_w // _CHUNK

    def step(g, carry):
        off = base + g * _CHUNK
        pltpu.sync_copy(idx_hbm.at[pl.ds(off, _CHUNK)], idx_v)
        pltpu.async_copy(table_hbm.at[idx_v], rows_v, sem).wait()
        pltpu.sync_copy(rows_v, out_hbm.at[pl.ds(off, _CHUNK)])
        return carry

    lax.fori_loop(0, nchunks, step, 0)


@functools.cache
def _make_emb(n_total: int):
    return functools.partial(
        pl.kernel,
        mesh=plsc.VectorSubcoreMesh(core_axis_name="c", subcore_axis_name="s"),
        out_type=jax.ShapeDtypeStruct((n_total, _DIM), jnp.float32),
        scratch_types=[
            pltpu.VMEM((_CHUNK,), jnp.int32),
            pltpu.VMEM((_CHUNK, _DIM), jnp.float32),
            pltpu.SemaphoreType.DMA,
        ],
        compiler_params=pltpu.CompilerParams(use_tc_tiling_on_sc=False),
    )(_emb_body)


def kernel(embedding, x):
    b, l = x.shape
    flat = x.reshape(b * l).astype(jnp.int32)
    out = _make_emb(b * l)(embedding, flat)
    return out.reshape(b, l, _DIM)


# speedup vs baseline: 1.0150x; 1.0150x over previous
import functools

import jax
import jax.numpy as jnp
from jax import lax
from jax.experimental import pallas as pl
from jax.experimental.pallas import tpu as pltpu
from jax.experimental.pallas import tpu_sc as plsc

_DIM = 64
_NC = 2
_NS = 16
_NW = _NC * _NS
_CHUNK = 512
_NBUF = 2


def _emb_body(table_hbm, idx_hbm, out_hbm, idx_v, rows0, rows1, sem0, sem1):
    wid = lax.axis_index("s") * _NC + lax.axis_index("c")
    b_per_w = idx_hbm.shape[0] // _NW
    base = wid * b_per_w
    nchunks = b_per_w // _CHUNK

    pltpu.sync_copy(idx_hbm.at[pl.ds(base, b_per_w)], idx_v)

    rows = (rows0, rows1)
    sems = (sem0, sem1)

    def gstart(g, b):
        pltpu.async_copy(
            table_hbm.at[idx_v.at[pl.ds(g * _CHUNK, _CHUNK)]], rows[b], sems[b]
        )

    def gwait(g, b):
        pltpu.make_async_copy(
            table_hbm.at[idx_v.at[pl.ds(g * _CHUNK, _CHUNK)]], rows[b], sems[b]
        ).wait()

    gstart(0, 0)

    def outer(i, carry):
        for b in range(_NBUF):
            g = _NBUF * i + b
            gwait(g, b)

            @pl.when(g + 1 < nchunks)
            def _():
                gstart(g + 1, (b + 1) % _NBUF)

            pltpu.sync_copy(rows[b], out_hbm.at[pl.ds(base + g * _CHUNK, _CHUNK)])
        return carry

    lax.fori_loop(0, nchunks // _NBUF, outer, 0)


@functools.cache
def _make_emb(n_total: int):
    b_per_w = n_total // _NW
    return functools.partial(
        pl.kernel,
        mesh=plsc.VectorSubcoreMesh(core_axis_name="c", subcore_axis_name="s"),
        out_type=jax.ShapeDtypeStruct((n_total, _DIM), jnp.float32),
        scratch_types=[
            pltpu.VMEM((b_per_w,), jnp.int32),
            pltpu.VMEM((_CHUNK, _DIM), jnp.float32),
            pltpu.VMEM((_CHUNK, _DIM), jnp.float32),
            pltpu.SemaphoreType.DMA,
            pltpu.SemaphoreType.DMA,
        ],
        compiler_params=pltpu.CompilerParams(use_tc_tiling_on_sc=False),
    )(_emb_body)


def kernel(embedding, x):
    b, l = x.shape
    flat = x.reshape(b * l).astype(jnp.int32)
    out = _make_emb(b * l)(embedding, flat)
    return out.reshape(b, l, _DIM)
